# baseline (speedup 1.0000x reference)
import functools
import math

import jax
import jax.numpy as jnp
from jax import lax
from jax.experimental import pallas as pl
from jax.experimental.pallas import tpu as pltpu
from jax.experimental.pallas import tpu_sc as plsc

NC, NS = 2, 16
NW = NC * NS
B, L, EMB = 4096, 20, 128
N = B * L

NCH = 4
ROWS_C = N // NCH
PER_W_C = ROWS_C // NW
CHUNK = 320
NCHUNK = PER_W_C // CHUNK

_sc_mesh = plsc.VectorSubcoreMesh(core_axis_name="c", subcore_axis_name="s")


def _make_sc_gather(c):
    @functools.partial(
        pl.kernel,
        mesh=_sc_mesh,
        out_type=jax.ShapeDtypeStruct((ROWS_C, EMB), jnp.float32),
        scratch_types=[
            pltpu.VMEM((CHUNK,), jnp.int32),
            pltpu.VMEM((CHUNK,), jnp.int32),
            pltpu.VMEM((CHUNK, EMB), jnp.float32),
            pltpu.VMEM((CHUNK, EMB), jnp.float32),
            pltpu.SemaphoreType.DMA,
            pltpu.SemaphoreType.DMA,
            pltpu.SemaphoreType.DMA,
            pltpu.SemaphoreType.DMA,
        ],
    )
    def sc_gather(idx_hbm, table_hbm, out_hbm, i0, i1, r0, r1, g0, g1, s0, s1):
        wid = lax.axis_index("s") * NC + lax.axis_index("c")
        base_in = c * ROWS_C + wid * PER_W_C
        base_out = wid * PER_W_C
        idx_v = (i0, i1)
        rows_v = (r0, r1)
        gsem = (g0, g1)
        ssem = (s0, s1)

        def start_gather(i, b):
            off = base_in + i * CHUNK
            pltpu.sync_copy(idx_hbm.at[pl.ds(off, CHUNK)], idx_v[b])
            return pltpu.async_copy(table_hbm.at[idx_v[b]], rows_v[b], gsem[b])

        gathers = [None, None]
        scatters = [None, None]
        gathers[0] = start_gather(0, 0)
        for i in range(NCHUNK):
            b = i % 2
            nb = 1 - b
            if i + 1 < NCHUNK:
                if scatters[nb] is not None:
                    scatters[nb].wait()
                    scatters[nb] = None
                gathers[nb] = start_gather(i + 1, nb)
            gathers[b].wait()
            off = base_out + i * CHUNK
            scatters[b] = pltpu.async_copy(
                rows_v[b], out_hbm.at[pl.ds(off, CHUNK)], ssem[b]
            )
        for sc in scatters:
            if sc is not None:
                sc.wait()

    return sc_gather


_sc_gathers = [_make_sc_gather(c) for c in range(NCH)]

BLK = 5120
NBLK_C = ROWS_C // BLK


def _gelu_mlp(x, w1, b1, w2, b2):
    h = jnp.dot(x, w1, preferred_element_type=jnp.float32) + b1
    h = 0.5 * h * (1.0 + lax.erf(h * (1.0 / math.sqrt(2.0))))
    return jnp.dot(h, w2, preferred_element_type=jnp.float32) + b2


def _mlp_first_body(x_ref, w1_ref, b1_ref, w2_ref, b2_ref, o_ref):
    o_ref[...] = _gelu_mlp(x_ref[...], w1_ref[...], b1_ref[...], w2_ref[...], b2_ref[...])


def _mlp_acc_body(acc_ref, x_ref, w1_ref, b1_ref, w2_ref, b2_ref, o_ref):
    del acc_ref
    o_ref[...] = _gelu_mlp(x_ref[...], w1_ref[...], b1_ref[...], w2_ref[...], b2_ref[...])


_w_specs = [
    pl.BlockSpec((EMB, EMB), lambda i: (0, 0)),
    pl.BlockSpec((1, EMB), lambda i: (0, 0)),
    pl.BlockSpec((EMB, EMB), lambda i: (0, 0)),
    pl.BlockSpec((1, EMB), lambda i: (0, 0)),
]


def _make_mlp(c):
    out_spec = pl.BlockSpec((BLK, EMB), lambda i, c=c: (c * NBLK_C + i, 0))
    x_spec = pl.BlockSpec((BLK, EMB), lambda i: (i, 0))
    if c == 0:
        return pl.pallas_call(
            _mlp_first_body,
            grid=(NBLK_C,),
            in_specs=[x_spec] + _w_specs,
            out_specs=out_spec,
            out_shape=jax.ShapeDtypeStruct((N, EMB), jnp.float32),
        )
    return pl.pallas_call(
        _mlp_acc_body,
        grid=(NBLK_C,),
        in_specs=[pl.BlockSpec(memory_space=pltpu.MemorySpace.HBM), x_spec] + _w_specs,
        out_specs=out_spec,
        out_shape=jax.ShapeDtypeStruct((N, EMB), jnp.float32),
        input_output_aliases={0: 0},
    )


_mlps = [_make_mlp(c) for c in range(NCH)]


def kernel(te, E, W1, b1, W2, b2):
    idx = te.T.reshape(-1).astype(jnp.int32)
    b1r = b1.reshape(1, EMB)
    b2r = b2.reshape(1, EMB)
    rows = [_sc_gathers[c](idx, E) for c in range(NCH)]
    out = _mlps[0](rows[0], W1, b1r, W2, b2r)
    for c in range(1, NCH):
        out = _mlps[c](out, rows[c], W1, b1r, W2, b2r)
    return out.reshape(L, B, EMB).transpose(1, 0, 2)


# speedup vs baseline: 1.0174x; 1.0174x over previous
import functools
import math

import jax
import jax.numpy as jnp
from jax import lax
from jax.experimental import pallas as pl
from jax.experimental.pallas import tpu as pltpu
from jax.experimental.pallas import tpu_sc as plsc

NC, NS = 2, 16
NW = NC * NS
B, L, EMB = 4096, 20, 128
N = B * L

NCH = 2
ROWS_C = N // NCH
PER_W_C = ROWS_C // NW
CHUNK = 320
NCHUNK = PER_W_C // CHUNK

_sc_mesh = plsc.VectorSubcoreMesh(core_axis_name="c", subcore_axis_name="s")


def _make_sc_gather(c):
    @functools.partial(
        pl.kernel,
        mesh=_sc_mesh,
        out_type=jax.ShapeDtypeStruct((ROWS_C, EMB), jnp.float32),
        scratch_types=[
            pltpu.VMEM((CHUNK,), jnp.int32),
            pltpu.VMEM((CHUNK,), jnp.int32),
            pltpu.VMEM((CHUNK, EMB), jnp.float32),
            pltpu.VMEM((CHUNK, EMB), jnp.float32),
            pltpu.SemaphoreType.DMA,
            pltpu.SemaphoreType.DMA,
            pltpu.SemaphoreType.DMA,
            pltpu.SemaphoreType.DMA,
        ],
    )
    def sc_gather(idx_hbm, table_hbm, out_hbm, i0, i1, r0, r1, g0, g1, s0, s1):
        wid = lax.axis_index("s") * NC + lax.axis_index("c")
        base_in = c * ROWS_C + wid * PER_W_C
        base_out = wid * PER_W_C
        idx_v = (i0, i1)
        rows_v = (r0, r1)
        gsem = (g0, g1)
        ssem = (s0, s1)

        def start_gather(i, b):
            off = base_in + i * CHUNK
            pltpu.sync_copy(idx_hbm.at[pl.ds(off, CHUNK)], idx_v[b])
            return pltpu.async_copy(table_hbm.at[idx_v[b]], rows_v[b], gsem[b])

        gathers = [None, None]
        scatters = [None, None]
        gathers[0] = start_gather(0, 0)
        for i in range(NCHUNK):
            b = i % 2
            nb = 1 - b
            if i + 1 < NCHUNK:
                if scatters[nb] is not None:
                    scatters[nb].wait()
                    scatters[nb] = None
                gathers[nb] = start_gather(i + 1, nb)
            gathers[b].wait()
            off = base_out + i * CHUNK
            scatters[b] = pltpu.async_copy(
                rows_v[b], out_hbm.at[pl.ds(off, CHUNK)], ssem[b]
            )
        for sc in scatters:
            if sc is not None:
                sc.wait()

    return sc_gather


_sc_gathers = [_make_sc_gather(c) for c in range(NCH)]

BLK = 8192
NBLK_C = ROWS_C // BLK


def _gelu_mlp(x, w1, b1, w2, b2):
    h = jnp.dot(x, w1, preferred_element_type=jnp.float32) + b1
    h = 0.5 * h * (1.0 + lax.erf(h * (1.0 / math.sqrt(2.0))))
    return jnp.dot(h, w2, preferred_element_type=jnp.float32) + b2


def _mlp_first_body(x_ref, w1_ref, b1_ref, w2_ref, b2_ref, o_ref):
    o_ref[...] = _gelu_mlp(x_ref[...], w1_ref[...], b1_ref[...], w2_ref[...], b2_ref[...])


def _mlp_acc_body(acc_ref, x_ref, w1_ref, b1_ref, w2_ref, b2_ref, o_ref):
    del acc_ref
    o_ref[...] = _gelu_mlp(x_ref[...], w1_ref[...], b1_ref[...], w2_ref[...], b2_ref[...])


_w_specs = [
    pl.BlockSpec((EMB, EMB), lambda i: (0, 0)),
    pl.BlockSpec((1, EMB), lambda i: (0, 0)),
    pl.BlockSpec((EMB, EMB), lambda i: (0, 0)),
    pl.BlockSpec((1, EMB), lambda i: (0, 0)),
]


def _make_mlp(c):
    out_spec = pl.BlockSpec((BLK, EMB), lambda i, c=c: (c * NBLK_C + i, 0))
    x_spec = pl.BlockSpec((BLK, EMB), lambda i: (i, 0))
    if c == 0:
        return pl.pallas_call(
            _mlp_first_body,
            grid=(NBLK_C,),
            in_specs=[x_spec] + _w_specs,
            out_specs=out_spec,
            out_shape=jax.ShapeDtypeStruct((N, EMB), jnp.float32),
        )
    return pl.pallas_call(
        _mlp_acc_body,
        grid=(NBLK_C,),
        in_specs=[pl.BlockSpec(memory_space=pltpu.MemorySpace.HBM), x_spec] + _w_specs,
        out_specs=out_spec,
        out_shape=jax.ShapeDtypeStruct((N, EMB), jnp.float32),
        input_output_aliases={0: 0},
    )


_mlps = [_make_mlp(c) for c in range(NCH)]


def kernel(te, E, W1, b1, W2, b2):
    idx = te.T.reshape(-1).astype(jnp.int32)
    b1r = b1.reshape(1, EMB)
    b2r = b2.reshape(1, EMB)
    rows = [_sc_gathers[c](idx, E) for c in range(NCH)]
    out = _mlps[0](rows[0], W1, b1r, W2, b2r)
    for c in range(1, NCH):
        out = _mlps[c](out, rows[c], W1, b1r, W2, b2r)
    return out.reshape(L, B, EMB).transpose(1, 0, 2)


# speedup vs baseline: 1.0827x; 1.0641x over previous
import functools
import math

import jax
import jax.numpy as jnp
from jax import lax
from jax.experimental import pallas as pl
from jax.experimental.pallas import tpu as pltpu
from jax.experimental.pallas import tpu_sc as plsc

NC, NS = 2, 16
NW = NC * NS
B, L, EMB = 4096, 20, 128
N = B * L

NCH = 1
ROWS_C = N // NCH
PER_W_C = ROWS_C // NW
CHUNK = 320
NCHUNK = PER_W_C // CHUNK

_sc_mesh = plsc.VectorSubcoreMesh(core_axis_name="c", subcore_axis_name="s")


def _make_sc_gather(c):
    @functools.partial(
        pl.kernel,
        mesh=_sc_mesh,
        out_type=jax.ShapeDtypeStruct((ROWS_C, EMB), jnp.float32),
        scratch_types=[
            pltpu.VMEM((PER_W_C,), jnp.int32),
            pltpu.VMEM((CHUNK, EMB), jnp.float32),
            pltpu.VMEM((CHUNK, EMB), jnp.float32),
            pltpu.VMEM((CHUNK, EMB), jnp.float32),
            pltpu.SemaphoreType.DMA,
            pltpu.SemaphoreType.DMA,
            pltpu.SemaphoreType.DMA,
            pltpu.SemaphoreType.DMA,
            pltpu.SemaphoreType.DMA,
            pltpu.SemaphoreType.DMA,
        ],
    )
    def sc_gather(idx_hbm, table_hbm, out_hbm, idx_all, r0, r1, r2,
                  g0, g1, g2, s0, s1, s2):
        wid = lax.axis_index("s") * NC + lax.axis_index("c")
        base_in = c * ROWS_C + wid * PER_W_C
        base_out = wid * PER_W_C
        rows_v = (r0, r1, r2)
        gsem = (g0, g1, g2)
        ssem = (s0, s1, s2)

        pltpu.sync_copy(idx_hbm.at[pl.ds(base_in, PER_W_C)], idx_all)

        def start_gather(i, b):
            idx_sl = idx_all.at[pl.ds(i * CHUNK, CHUNK)]
            return pltpu.async_copy(table_hbm.at[idx_sl], rows_v[b], gsem[b])

        gathers = [None, None, None]
        scatters = [None, None, None]
        for b in range(min(2, NCHUNK)):
            gathers[b] = start_gather(b, b)
        for i in range(NCHUNK):
            b = i % 3
            j = i + 2
            if j < NCHUNK:
                bj = j % 3
                if scatters[bj] is not None:
                    scatters[bj].wait()
                    scatters[bj] = None
                gathers[bj] = start_gather(j, bj)
            gathers[b].wait()
            scatters[b] = pltpu.async_copy(
                rows_v[b], out_hbm.at[pl.ds(base_out + i * CHUNK, CHUNK)], ssem[b]
            )
        for sc in scatters:
            if sc is not None:
                sc.wait()

    return sc_gather


_sc_gathers = [_make_sc_gather(c) for c in range(NCH)]

BLK = 16384
NBLK_C = ROWS_C // BLK


def _gelu_mlp(x, w1, b1, w2, b2):
    h = jnp.dot(x, w1, preferred_element_type=jnp.float32) + b1
    h = 0.5 * h * (1.0 + lax.erf(h * (1.0 / math.sqrt(2.0))))
    return jnp.dot(h, w2, preferred_element_type=jnp.float32) + b2


def _mlp_first_body(x_ref, w1_ref, b1_ref, w2_ref, b2_ref, o_ref):
    o_ref[...] = _gelu_mlp(x_ref[...], w1_ref[...], b1_ref[...], w2_ref[...], b2_ref[...])


def _mlp_acc_body(acc_ref, x_ref, w1_ref, b1_ref, w2_ref, b2_ref, o_ref):
    del acc_ref
    o_ref[...] = _gelu_mlp(x_ref[...], w1_ref[...], b1_ref[...], w2_ref[...], b2_ref[...])


_w_specs = [
    pl.BlockSpec((EMB, EMB), lambda i: (0, 0)),
    pl.BlockSpec((1, EMB), lambda i: (0, 0)),
    pl.BlockSpec((EMB, EMB), lambda i: (0, 0)),
    pl.BlockSpec((1, EMB), lambda i: (0, 0)),
]


def _make_mlp(c):
    out_spec = pl.BlockSpec((BLK, EMB), lambda i, c=c: (c * NBLK_C + i, 0))
    x_spec = pl.BlockSpec((BLK, EMB), lambda i: (i, 0))
    if c == 0:
        return pl.pallas_call(
            _mlp_first_body,
            grid=(NBLK_C,),
            in_specs=[x_spec] + _w_specs,
            out_specs=out_spec,
            out_shape=jax.ShapeDtypeStruct((N, EMB), jnp.float32),
        )
    return pl.pallas_call(
        _mlp_acc_body,
        grid=(NBLK_C,),
        in_specs=[pl.BlockSpec(memory_space=pltpu.MemorySpace.HBM), x_spec] + _w_specs,
        out_specs=out_spec,
        out_shape=jax.ShapeDtypeStruct((N, EMB), jnp.float32),
        input_output_aliases={0: 0},
    )


_mlps = [_make_mlp(c) for c in range(NCH)]


def kernel(te, E, W1, b1, W2, b2):
    idx = te.T.reshape(-1).astype(jnp.int32)
    b1r = b1.reshape(1, EMB)
    b2r = b2.reshape(1, EMB)
    rows = [_sc_gathers[c](idx, E) for c in range(NCH)]
    out = _mlps[0](rows[0], W1, b1r, W2, b2r)
    for c in range(1, NCH):
        out = _mlps[c](out, rows[c], W1, b1r, W2, b2r)
    return out.reshape(L, B, EMB).transpose(1, 0, 2)
